# Initial kernel scaffold; baseline (speedup 1.0000x reference)
import jax
import jax.numpy as jnp
from jax.experimental import pallas as pl


def kernel(atom_features, atom_split, U, b):
    raise NotImplementedError("write your pallas kernel here")


# speedup vs baseline: 2.9501x; 2.9501x over previous
import functools

import jax
import jax.numpy as jnp
from jax import lax
from jax.experimental import pallas as pl
from jax.experimental.pallas import tpu as pltpu
from jax.experimental.pallas import tpu_sc as plsc

HID = 128
NMOL = 4096
STEPS = 6

NC, NS, L = 2, 16, 16
NW = NC * NS
N_PAD = 100352
APT = N_PAD // NW
BLK = 112
NBLK = APT // BLK
NGRP = BLK // L
ACC_ROWS = 4352
STRIPE = ACC_ROWS // NS
H_PAD_ROWS = 4104

_sc_mesh = plsc.VectorSubcoreMesh(
    core_axis_name="c", subcore_axis_name="s", num_cores=NC, num_subcores=NS)


def _attn_body(x_hbm, seg_hbm, h_hbm, num_hbm, den_hbm,
               seg_v, x_v, h_v, o_v, zv, den_v, bf_v, acc, sem):
    c = lax.axis_index("c")
    s = lax.axis_index("s")

    zero16 = jnp.zeros((L,), jnp.float32)

    def zrow(i, _):
        for k in range(HID // L):
            zv[i, pl.ds(L * k, L)] = zero16
        return 0
    lax.fori_loop(0, L, zrow, 0)

    def zacc(j, _):
        pltpu.sync_copy(zv, acc.at[pl.ds(s * STRIPE + L * j, L)])
        return 0
    lax.fori_loop(0, STRIPE // L, zacc, 0)

    def zden(j, _):
        den_v[pl.ds(L * j, L)] = zero16
        return 0
    lax.fori_loop(0, ACC_ROWS // L, zden, 0)
    plsc.subcore_barrier()

    wid = s * NC + c
    base = wid * APT
    lanes = lax.iota(jnp.int32, L)
    onehots = [(lanes == j).astype(jnp.float32) for j in range(L)]

    def blk_body(bi, _):
        off = base + bi * BLK
        pltpu.sync_copy(seg_hbm.at[pl.ds(off, BLK)], seg_v)
        pltpu.sync_copy(x_hbm.at[pl.ds(off, BLK)], x_v)
        pltpu.async_copy(h_hbm.at[seg_v], h_v, sem).wait()

        def grp(g, _):
            seg16 = seg_v[pl.ds(g * L, L)]
            for j in range(L):
                a = g * L + j
                xs = []
                acc16 = zero16
                for k in range(HID // L):
                    xk = x_v[a, pl.ds(L * k, L)]
                    hk = h_v[a, pl.ds(L * k, L)]
                    xs.append(xk)
                    acc16 = acc16 + xk * hk
                v = acc16
                for m in (8, 4, 2, 1):
                    bf_v[...] = v
                    v = v + plsc.load_gather(bf_v, [lanes ^ m])
                w16 = jnp.exp(v)
                for k in range(HID // L):
                    o_v[a, pl.ds(L * k, L)] = w16 * xs[k]
                plsc.addupdate_scatter(den_v, [seg16], w16 * onehots[j])
            return 0
        lax.fori_loop(0, NGRP, grp, 0)

        pltpu.sync_copy(o_v, acc.at[seg_v], add=True)
        return 0
    lax.fori_loop(0, NBLK, blk_body, 0)
    plsc.subcore_barrier()

    row0 = s * STRIPE
    pltpu.sync_copy(acc.at[pl.ds(row0, STRIPE)],
                    num_hbm.at[pl.ds(c * ACC_ROWS + row0, STRIPE)])
    pltpu.sync_copy(den_v, den_hbm.at[wid])


_attn = functools.partial(
    pl.kernel,
    out_type=(
        jax.ShapeDtypeStruct((NC * ACC_ROWS, HID), jnp.float32),
        jax.ShapeDtypeStruct((NW, ACC_ROWS), jnp.float32),
    ),
    mesh=_sc_mesh,
    compiler_params=pltpu.CompilerParams(needs_layout_passes=False),
    scratch_types=[
        pltpu.VMEM((BLK,), jnp.int32),
        pltpu.VMEM((BLK, HID), jnp.float32),
        pltpu.VMEM((BLK, HID), jnp.float32),
        pltpu.VMEM((BLK, HID), jnp.float32),
        pltpu.VMEM((L, HID), jnp.float32),
        pltpu.VMEM((ACC_ROWS,), jnp.float32),
        pltpu.VMEM((L,), jnp.float32),
        pltpu.VMEM_SHARED((ACC_ROWS, HID), jnp.float32),
        pltpu.SemaphoreType.DMA,
    ],
)(_attn_body)


def _lstm_body(h_ref, c_ref, num_ref, den_ref, u_ref, b_ref, q_ref, h_out, c_out):
    num = num_ref[0] + num_ref[1]
    den = jnp.sum(den_ref[...], axis=0)
    rinv = jnp.where(den > 0, 1.0 / den, 0.0)
    r = num * rinv[:, None]
    h = h_ref[...]
    q = jnp.concatenate([h, r], axis=1)
    q_ref[...] = q
    z = jnp.dot(q, u_ref[...], preferred_element_type=jnp.float32) + b_ref[...]
    i = jax.nn.sigmoid(z[:, :HID])
    f = jax.nn.sigmoid(z[:, HID:2 * HID])
    o = jax.nn.sigmoid(z[:, 2 * HID:3 * HID])
    g = z[:, 3 * HID:]
    c_new = f * c_ref[...] + i * jnp.tanh(g)
    h_out[...] = o * jnp.tanh(c_new)
    c_out[...] = c_new


_ROWS_BLK = 256
_lstm = pl.pallas_call(
    _lstm_body,
    grid=(NMOL // _ROWS_BLK,),
    in_specs=[
        pl.BlockSpec((_ROWS_BLK, HID), lambda i: (i, 0)),
        pl.BlockSpec((_ROWS_BLK, HID), lambda i: (i, 0)),
        pl.BlockSpec((2, _ROWS_BLK, HID), lambda i: (0, i, 0)),
        pl.BlockSpec((NW, _ROWS_BLK), lambda i: (0, i)),
        pl.BlockSpec((2 * HID, 4 * HID), lambda i: (0, 0)),
        pl.BlockSpec((1, 4 * HID), lambda i: (0, 0)),
    ],
    out_specs=[
        pl.BlockSpec((_ROWS_BLK, 2 * HID), lambda i: (i, 0)),
        pl.BlockSpec((_ROWS_BLK, HID), lambda i: (i, 0)),
        pl.BlockSpec((_ROWS_BLK, HID), lambda i: (i, 0)),
    ],
    out_shape=[
        jax.ShapeDtypeStruct((NMOL, 2 * HID), jnp.float32),
        jax.ShapeDtypeStruct((NMOL, HID), jnp.float32),
        jax.ShapeDtypeStruct((NMOL, HID), jnp.float32),
    ],
)


def kernel(atom_features, atom_split, U, b):
    n = atom_features.shape[0]
    seg = atom_split.astype(jnp.int32)
    xp = jnp.concatenate(
        [atom_features, jnp.zeros((N_PAD - n, HID), jnp.float32)], axis=0)
    segp = jnp.concatenate(
        [seg, jnp.full((N_PAD - n,), NMOL, jnp.int32)], axis=0)
    b2 = b.reshape(1, 4 * HID)

    h = jnp.zeros((NMOL, HID), jnp.float32)
    c = jnp.zeros((NMOL, HID), jnp.float32)
    q = None
    for _ in range(STEPS):
        hp = jnp.concatenate(
            [h, jnp.zeros((H_PAD_ROWS - NMOL, HID), jnp.float32)], axis=0)
        num, den = _attn(xp, segp, hp)
        nump = num.reshape(NC, ACC_ROWS, HID)[:, :NMOL, :]
        denp = den[:, :NMOL]
        q, h, c = _lstm(h, c, nump, denp, U, b2)
    return q
